# baseline (speedup 1.0000x reference)
import functools

import jax
import jax.numpy as jnp
from jax import lax
from jax.experimental import pallas as pl
from jax.experimental.pallas import tpu as pltpu
from jax.experimental.pallas import tpu_sc as plsc

N_NODES = 10000
N_EDGES = 320000
D = 128
DW = D // 2

NC = 2
NS = 16
NW = NC * NS
EPW = N_EDGES // NW
C = 80
NCHUNK = EPW // C

_mesh = plsc.VectorSubcoreMesh(
    core_axis_name="c", subcore_axis_name="s", num_cores=NC, num_subcores=NS
)


def _rsqrt_nr(s):
    y = plsc.bitcast(jnp.int32(0x5F3759DF) - (plsc.bitcast(s, jnp.int32) >> 1),
                     jnp.float32)
    h = 0.5 * s
    y = y * (1.5 - h * y * y)
    y = y * (1.5 - h * y * y)
    y = y * (1.5 - h * y * y)
    return y


@functools.partial(
    pl.kernel,
    out_type=jax.ShapeDtypeStruct((N_EDGES,), jnp.float32),
    mesh=_mesh,
    compiler_params=pltpu.CompilerParams(needs_layout_passes=False,
                                         use_tc_tiling_on_sc=False),
    scratch_types=[
        pltpu.VMEM((EPW,), jnp.int32),
        pltpu.VMEM((EPW,), jnp.int32),
        pltpu.VMEM((EPW,), jnp.float32),
        pltpu.VMEM((EPW,), jnp.float32),
        pltpu.VMEM((C, DW), jnp.int32),
        pltpu.VMEM((C, DW), jnp.int32),
        pltpu.VMEM((C, DW), jnp.int32),
        pltpu.VMEM((C, DW), jnp.int32),
        pltpu.SemaphoreType.DMA,
        pltpu.SemaphoreType.DMA,
    ],
)
def _score_kernel(head_hbm, tail_hbm, x_hbm, sw_hbm, out_hbm,
                  hidx_v, tidx_v, sw_v, score_v,
                  hr_a, tr_a, hr_b, tr_b, sem_a, sem_b):
    wid = lax.axis_index("s") * NC + lax.axis_index("c")
    base = pl.multiple_of(wid * EPW, EPW)

    pltpu.sync_copy(head_hbm.at[pl.ds(base, EPW)], hidx_v)
    pltpu.sync_copy(tail_hbm.at[pl.ds(base, EPW)], tidx_v)
    pltpu.sync_copy(sw_hbm.at[pl.ds(base, EPW)], sw_v)

    lane = lax.iota(jnp.int32, 16)

    def mk_gathers(g, hr, tr, sem):
        off = pl.multiple_of(g * C, C)
        ch = pltpu.make_async_copy(x_hbm.at[hidx_v.at[pl.ds(off, C)]], hr, sem)
        ct = pltpu.make_async_copy(x_hbm.at[tidx_v.at[pl.ds(off, C)]], tr, sem)
        return ch, ct

    def start(g, hr, tr, sem):
        ch, ct = mk_gathers(g, hr, tr, sem)
        ch.start()
        ct.start()

    def wait(g, hr, tr, sem):
        ch, ct = mk_gathers(g, hr, tr, sem)
        ch.wait()
        ct.wait()

    def compute(g, hr, tr):
        cbase = pl.multiple_of(g * C, C)
        v = plsc.bitcast(hr[0, pl.ds(0, 16)], jnp.float32)
        score_v[pl.ds(cbase, 16)] = v

    start(0, hr_a, tr_a, sem_a)
    start(1, hr_b, tr_b, sem_b)

    def pair_body(k, carry):
        g = 2 * k
        wait(g, hr_a, tr_a, sem_a)
        compute(g, hr_a, tr_a)
        start(g + 2, hr_a, tr_a, sem_a)
        wait(g + 1, hr_b, tr_b, sem_b)
        compute(g + 1, hr_b, tr_b)

        @pl.when(k < NCHUNK // 2 - 1)
        def _():
            start(g + 3, hr_b, tr_b, sem_b)

        return carry

    lax.fori_loop(0, NCHUNK // 2, pair_body, 0)
    wait(NCHUNK - 1, hr_a, tr_a, sem_a)
    compute(NCHUNK - 1, hr_a, tr_a)

    pltpu.sync_copy(score_v, out_hbm.at[pl.ds(base, EPW)])


def kernel(tuples, x, sw):
    head = tuples[:, 0]
    tail = tuples[:, 1]
    u = jax.lax.bitcast_convert_type(x.astype(jnp.bfloat16), jnp.uint16)
    w = u.astype(jnp.uint32)
    xw = jax.lax.bitcast_convert_type(w[:, :DW] | (w[:, DW:] << 16),
                                      jnp.int32)
    return _score_kernel(head, tail, xw, sw)


# speedup vs baseline: 1.0149x; 1.0149x over previous
import functools

import jax
import jax.numpy as jnp
from jax import lax
from jax.experimental import pallas as pl
from jax.experimental.pallas import tpu as pltpu
from jax.experimental.pallas import tpu_sc as plsc

N_NODES = 10000
N_EDGES = 320000
D = 128
DW = D // 2

NC = 2
NS = 16
NW = NC * NS
EPW = N_EDGES // NW
C = 400
NCHUNK = EPW // C

_mesh = plsc.VectorSubcoreMesh(
    core_axis_name="c", subcore_axis_name="s", num_cores=NC, num_subcores=NS
)


def _rsqrt_nr(s):
    y = plsc.bitcast(jnp.int32(0x5F3759DF) - (plsc.bitcast(s, jnp.int32) >> 1),
                     jnp.float32)
    h = 0.5 * s
    y = y * (1.5 - h * y * y)
    y = y * (1.5 - h * y * y)
    y = y * (1.5 - h * y * y)
    return y


@functools.partial(
    pl.kernel,
    out_type=jax.ShapeDtypeStruct((N_EDGES,), jnp.float32),
    mesh=_mesh,
    compiler_params=pltpu.CompilerParams(needs_layout_passes=False,
                                         use_tc_tiling_on_sc=False),
    scratch_types=[
        pltpu.VMEM((EPW,), jnp.int32),
        pltpu.VMEM((EPW,), jnp.int32),
        pltpu.VMEM((C, DW), jnp.int32),
        pltpu.VMEM((C, DW), jnp.int32),
        pltpu.VMEM((C, DW), jnp.int32),
        pltpu.VMEM((C, DW), jnp.int32),
        pltpu.VMEM((C,), jnp.float32),
        pltpu.VMEM((C,), jnp.float32),
        pltpu.VMEM((C,), jnp.float32),
        pltpu.VMEM((C,), jnp.float32),
        pltpu.SemaphoreType.DMA,
        pltpu.SemaphoreType.DMA,
        pltpu.SemaphoreType.DMA,
        pltpu.SemaphoreType.DMA,
    ],
)
def _score_kernel(head_hbm, tail_hbm, x_hbm, sw_hbm, out_hbm,
                  hidx_v, tidx_v, hr_a, tr_a, hr_b, tr_b,
                  sw_a, sw_b, sc_a, sc_b, sem_a, sem_b, semw_a, semw_b):
    wid = lax.axis_index("s") * NC + lax.axis_index("c")
    base = pl.multiple_of(wid * EPW, EPW)

    pltpu.sync_copy(head_hbm.at[pl.ds(base, EPW)], hidx_v)
    pltpu.sync_copy(tail_hbm.at[pl.ds(base, EPW)], tidx_v)

    lane = lax.iota(jnp.int32, 16)

    def mk_in(g, hr, tr, swb, sem):
        off = pl.multiple_of(g * C, C)
        ch = pltpu.make_async_copy(x_hbm.at[hidx_v.at[pl.ds(off, C)]], hr, sem)
        ct = pltpu.make_async_copy(x_hbm.at[tidx_v.at[pl.ds(off, C)]], tr, sem)
        cs = pltpu.make_async_copy(sw_hbm.at[pl.ds(base + off, C)], swb, sem)
        return ch, ct, cs

    def start(g, hr, tr, swb, sem):
        for c in mk_in(g, hr, tr, swb, sem):
            c.start()

    def wait(g, hr, tr, swb, sem):
        for c in mk_in(g, hr, tr, swb, sem):
            c.wait()

    def mk_wr(g, scb, semw):
        off = pl.multiple_of(g * C, C)
        return pltpu.make_async_copy(scb, out_hbm.at[pl.ds(base + off, C)],
                                     semw)

    def compute(g, hr, tr, swb, scb):
        def grp_body(kk, c2):
            rbase = kk * 16
            ssvec = jnp.zeros((16,), jnp.float32)
            for i in range(16):
                e = rbase + i
                acc = jnp.zeros((16,), jnp.float32)
                for j in range(DW // 16):
                    h = plsc.bitcast(hr[e, pl.ds(j * 16, 16)], jnp.bfloat16)
                    t = plsc.bitcast(tr[e, pl.ds(j * 16, 16)], jnp.bfloat16)
                    d = h - t
                    sq_a, sq_b = plsc.unpack(d * d,
                                             format=plsc.PackFormat.INTERLEAVED)
                    acc = acc + sq_a + sq_b
                ssvec = jnp.where(lane == i, jnp.sum(acc), ssvec)
            y = _rsqrt_nr(jnp.maximum(ssvec, 1e-12))
            sl = pl.ds(rbase, 16)
            scb[sl] = ssvec * y * swb[sl]
            return c2

        lax.fori_loop(0, C // 16, grp_body, 0)

    start(0, hr_a, tr_a, sw_a, sem_a)
    start(1, hr_b, tr_b, sw_b, sem_b)

    def pair_body(k, carry):
        g = 2 * k
        wait(g, hr_a, tr_a, sw_a, sem_a)

        @pl.when(k > 0)
        def _():
            mk_wr(g - 2, sc_a, semw_a).wait()

        compute(g, hr_a, tr_a, sw_a, sc_a)
        mk_wr(g, sc_a, semw_a).start()
        start(g + 2, hr_a, tr_a, sw_a, sem_a)

        wait(g + 1, hr_b, tr_b, sw_b, sem_b)

        @pl.when(k > 0)
        def _():
            mk_wr(g - 1, sc_b, semw_b).wait()

        compute(g + 1, hr_b, tr_b, sw_b, sc_b)
        mk_wr(g + 1, sc_b, semw_b).start()

        @pl.when(k < NCHUNK // 2 - 1)
        def _():
            start(g + 3, hr_b, tr_b, sw_b, sem_b)

        return carry

    lax.fori_loop(0, NCHUNK // 2, pair_body, 0)
    wait(NCHUNK - 1, hr_a, tr_a, sw_a, sem_a)
    mk_wr(NCHUNK - 3, sc_a, semw_a).wait()
    compute(NCHUNK - 1, hr_a, tr_a, sw_a, sc_a)
    mk_wr(NCHUNK - 1, sc_a, semw_a).start()
    mk_wr(NCHUNK - 2, sc_b, semw_b).wait()
    mk_wr(NCHUNK - 1, sc_a, semw_a).wait()


def kernel(tuples, x, sw):
    head = tuples[:, 0]
    tail = tuples[:, 1]
    u = jax.lax.bitcast_convert_type(x.astype(jnp.bfloat16), jnp.uint16)
    w = u.astype(jnp.uint32)
    xw = jax.lax.bitcast_convert_type(w[:, :DW] | (w[:, DW:] << 16),
                                      jnp.int32)
    return _score_kernel(head, tail, xw, sw)


# speedup vs baseline: 1.0155x; 1.0006x over previous
import functools

import jax
import jax.numpy as jnp
from jax import lax
from jax.experimental import pallas as pl
from jax.experimental.pallas import tpu as pltpu
from jax.experimental.pallas import tpu_sc as plsc

N_NODES = 10000
N_EDGES = 320000
D = 128
DW = D // 2

NC = 2
NS = 16
NW = NC * NS
EPW = N_EDGES // NW
C = 400
NCHUNK = EPW // C

_mesh = plsc.VectorSubcoreMesh(
    core_axis_name="c", subcore_axis_name="s", num_cores=NC, num_subcores=NS
)


def _rsqrt_nr(s):
    y = plsc.bitcast(jnp.int32(0x5F3759DF) - (plsc.bitcast(s, jnp.int32) >> 1),
                     jnp.float32)
    h = 0.5 * s
    y = y * (1.5 - h * y * y)
    y = y * (1.5 - h * y * y)
    y = y * (1.5 - h * y * y)
    return y


@functools.partial(
    pl.kernel,
    out_type=jax.ShapeDtypeStruct((N_EDGES,), jnp.float32),
    mesh=_mesh,
    compiler_params=pltpu.CompilerParams(needs_layout_passes=False,
                                         use_tc_tiling_on_sc=False),
    scratch_types=[
        pltpu.VMEM((EPW,), jnp.int32),
        pltpu.VMEM((EPW,), jnp.int32),
        pltpu.VMEM((C, DW), jnp.int32),
        pltpu.VMEM((C, DW), jnp.int32),
        pltpu.VMEM((C, DW), jnp.int32),
        pltpu.VMEM((C, DW), jnp.int32),
        pltpu.VMEM((C,), jnp.float32),
        pltpu.VMEM((C,), jnp.float32),
        pltpu.VMEM((C,), jnp.float32),
        pltpu.VMEM((C,), jnp.float32),
        pltpu.SemaphoreType.DMA,
        pltpu.SemaphoreType.DMA,
        pltpu.SemaphoreType.DMA,
        pltpu.SemaphoreType.DMA,
    ],
)
def _score_kernel(head_hbm, tail_hbm, x_hbm, sw_hbm, out_hbm,
                  hidx_v, tidx_v, hr_a, tr_a, hr_b, tr_b,
                  sw_a, sw_b, sc_a, sc_b, sem_a, sem_b, semw_a, semw_b):
    wid = lax.axis_index("s") * NC + lax.axis_index("c")
    base = pl.multiple_of(wid * EPW, EPW)

    pltpu.sync_copy(head_hbm.at[pl.ds(base, EPW)], hidx_v)
    pltpu.sync_copy(tail_hbm.at[pl.ds(base, EPW)], tidx_v)

    lane = lax.iota(jnp.int32, 16)

    def mk_in(g, hr, tr, swb, sem):
        off = pl.multiple_of(g * C, C)
        ch = pltpu.make_async_copy(x_hbm.at[hidx_v.at[pl.ds(off, C)]], hr, sem)
        ct = pltpu.make_async_copy(x_hbm.at[tidx_v.at[pl.ds(off, C)]], tr, sem)
        cs = pltpu.make_async_copy(sw_hbm.at[pl.ds(base + off, C)], swb, sem)
        return ch, ct, cs

    def start(g, hr, tr, swb, sem):
        for c in mk_in(g, hr, tr, swb, sem):
            c.start()

    def wait(g, hr, tr, swb, sem):
        for c in mk_in(g, hr, tr, swb, sem):
            c.wait()

    def mk_wr(g, scb, semw):
        off = pl.multiple_of(g * C, C)
        return pltpu.make_async_copy(scb, out_hbm.at[pl.ds(base + off, C)],
                                     semw)

    def compute(g, hr, tr, swb, scb):
        def grp_body(kk, c2):
            rbase = kk * 16
            ssvec = jnp.zeros((16,), jnp.float32)
            for i in range(16):
                e = rbase + i
                acc = jnp.zeros((16,), jnp.float32)
                for j in range(DW // 16):
                    h = plsc.bitcast(hr[e, pl.ds(j * 16, 16)], jnp.bfloat16)
                    t = plsc.bitcast(tr[e, pl.ds(j * 16, 16)], jnp.bfloat16)
                    d = h - t
                    sq_a, sq_b = plsc.unpack(d * d,
                                             format=plsc.PackFormat.INTERLEAVED)
                    acc = acc + sq_a + sq_b
                ssvec = jnp.where(lane == i, jnp.sum(acc), ssvec)
            y = _rsqrt_nr(jnp.maximum(ssvec, 1e-12))
            sl = pl.ds(rbase, 16)
            scb[sl] = ssvec * y * swb[sl]
            return c2

        lax.fori_loop(0, C // 16, grp_body, 0)

    start(0, hr_a, tr_a, sw_a, sem_a)
    start(1, hr_b, tr_b, sw_b, sem_b)

    def pair_body(k, carry):
        g = 2 * k
        wait(g, hr_a, tr_a, sw_a, sem_a)

        @pl.when(k > 0)
        def _():
            mk_wr(g - 2, sc_a, semw_a).wait()

        compute(g, hr_a, tr_a, sw_a, sc_a)
        mk_wr(g, sc_a, semw_a).start()
        start(g + 2, hr_a, tr_a, sw_a, sem_a)

        wait(g + 1, hr_b, tr_b, sw_b, sem_b)

        @pl.when(k > 0)
        def _():
            mk_wr(g - 1, sc_b, semw_b).wait()

        compute(g + 1, hr_b, tr_b, sw_b, sc_b)
        mk_wr(g + 1, sc_b, semw_b).start()

        @pl.when(k < NCHUNK // 2 - 1)
        def _():
            start(g + 3, hr_b, tr_b, sw_b, sem_b)

        return carry

    lax.fori_loop(0, NCHUNK // 2, pair_body, 0)
    wait(NCHUNK - 1, hr_a, tr_a, sw_a, sem_a)
    mk_wr(NCHUNK - 3, sc_a, semw_a).wait()
    compute(NCHUNK - 1, hr_a, tr_a, sw_a, sc_a)
    mk_wr(NCHUNK - 1, sc_a, semw_a).start()
    mk_wr(NCHUNK - 2, sc_b, semw_b).wait()
    mk_wr(NCHUNK - 1, sc_a, semw_a).wait()


def kernel(tuples, x, sw):
    tt = tuples.T
    head = tt[0]
    tail = tt[1]
    u = jax.lax.bitcast_convert_type(x.astype(jnp.bfloat16), jnp.uint16)
    w = u.astype(jnp.uint32)
    xw = jax.lax.bitcast_convert_type(w[:, :DW] | (w[:, DW:] << 16),
                                      jnp.int32)
    return _score_kernel(head, tail, xw, sw)


# speedup vs baseline: 1.0865x; 1.0700x over previous
import functools

import jax
import jax.numpy as jnp
from jax import lax
from jax.experimental import pallas as pl
from jax.experimental.pallas import tpu as pltpu
from jax.experimental.pallas import tpu_sc as plsc

N_NODES = 10000
N_EDGES = 320000
D = 128
DW = D // 2

NC = 2
NS = 16
NW = NC * NS
EPW = N_EDGES // NW
C = 400
NCHUNK = EPW // C

_mesh = plsc.VectorSubcoreMesh(
    core_axis_name="c", subcore_axis_name="s", num_cores=NC, num_subcores=NS
)


def _rsqrt_nr(s):
    y = plsc.bitcast(jnp.int32(0x5F3759DF) - (plsc.bitcast(s, jnp.int32) >> 1),
                     jnp.float32)
    h = 0.5 * s
    y = y * (1.5 - h * y * y)
    y = y * (1.5 - h * y * y)
    y = y * (1.5 - h * y * y)
    return y


@functools.partial(
    pl.kernel,
    out_type=jax.ShapeDtypeStruct((N_EDGES,), jnp.float32),
    mesh=_mesh,
    compiler_params=pltpu.CompilerParams(needs_layout_passes=False,
                                         use_tc_tiling_on_sc=False),
    scratch_types=[
        pltpu.VMEM((EPW,), jnp.int32),
        pltpu.VMEM((EPW,), jnp.int32),
        pltpu.VMEM((C, DW), jnp.int32),
        pltpu.VMEM((C, DW), jnp.int32),
        pltpu.VMEM((C, DW), jnp.int32),
        pltpu.VMEM((C, DW), jnp.int32),
        pltpu.VMEM((C,), jnp.float32),
        pltpu.VMEM((C,), jnp.float32),
        pltpu.VMEM((C,), jnp.float32),
        pltpu.VMEM((C,), jnp.float32),
        pltpu.SemaphoreType.DMA,
        pltpu.SemaphoreType.DMA,
        pltpu.SemaphoreType.DMA,
        pltpu.SemaphoreType.DMA,
    ],
)
def _score_kernel(tt_hbm, x_hbm, sw_hbm, out_hbm,
                  hidx_v, tidx_v, hr_a, tr_a, hr_b, tr_b,
                  sw_a, sw_b, sc_a, sc_b, sem_a, sem_b, semw_a, semw_b):
    wid = lax.axis_index("s") * NC + lax.axis_index("c")
    base = pl.multiple_of(wid * EPW, EPW)

    pltpu.sync_copy(tt_hbm.at[0, pl.ds(base, EPW)], hidx_v)
    pltpu.sync_copy(tt_hbm.at[1, pl.ds(base, EPW)], tidx_v)

    lane = lax.iota(jnp.int32, 16)

    def mk_in(g, hr, tr, swb, sem):
        off = pl.multiple_of(g * C, C)
        ch = pltpu.make_async_copy(x_hbm.at[hidx_v.at[pl.ds(off, C)]], hr, sem)
        ct = pltpu.make_async_copy(x_hbm.at[tidx_v.at[pl.ds(off, C)]], tr, sem)
        cs = pltpu.make_async_copy(sw_hbm.at[pl.ds(base + off, C)], swb, sem)
        return ch, ct, cs

    def start(g, hr, tr, swb, sem):
        for c in mk_in(g, hr, tr, swb, sem):
            c.start()

    def wait(g, hr, tr, swb, sem):
        for c in mk_in(g, hr, tr, swb, sem):
            c.wait()

    def mk_wr(g, scb, semw):
        off = pl.multiple_of(g * C, C)
        return pltpu.make_async_copy(scb, out_hbm.at[pl.ds(base + off, C)],
                                     semw)

    def compute(g, hr, tr, swb, scb):
        def grp_body(kk, c2):
            rbase = kk * 16
            ssvec = jnp.zeros((16,), jnp.float32)
            for i in range(16):
                e = rbase + i
                acc = jnp.zeros((16,), jnp.float32)
                for j in range(DW // 16):
                    h = plsc.bitcast(hr[e, pl.ds(j * 16, 16)], jnp.bfloat16)
                    t = plsc.bitcast(tr[e, pl.ds(j * 16, 16)], jnp.bfloat16)
                    d = h - t
                    sq_a, sq_b = plsc.unpack(d * d,
                                             format=plsc.PackFormat.INTERLEAVED)
                    acc = acc + sq_a + sq_b
                ssvec = jnp.where(lane == i, jnp.sum(acc), ssvec)
            y = _rsqrt_nr(jnp.maximum(ssvec, 1e-12))
            sl = pl.ds(rbase, 16)
            scb[sl] = ssvec * y * swb[sl]
            return c2

        lax.fori_loop(0, C // 16, grp_body, 0)

    start(0, hr_a, tr_a, sw_a, sem_a)
    start(1, hr_b, tr_b, sw_b, sem_b)

    def pair_body(k, carry):
        g = 2 * k
        wait(g, hr_a, tr_a, sw_a, sem_a)

        @pl.when(k > 0)
        def _():
            mk_wr(g - 2, sc_a, semw_a).wait()

        compute(g, hr_a, tr_a, sw_a, sc_a)
        mk_wr(g, sc_a, semw_a).start()
        start(g + 2, hr_a, tr_a, sw_a, sem_a)

        wait(g + 1, hr_b, tr_b, sw_b, sem_b)

        @pl.when(k > 0)
        def _():
            mk_wr(g - 1, sc_b, semw_b).wait()

        compute(g + 1, hr_b, tr_b, sw_b, sc_b)
        mk_wr(g + 1, sc_b, semw_b).start()

        @pl.when(k < NCHUNK // 2 - 1)
        def _():
            start(g + 3, hr_b, tr_b, sw_b, sem_b)

        return carry

    lax.fori_loop(0, NCHUNK // 2, pair_body, 0)
    wait(NCHUNK - 1, hr_a, tr_a, sw_a, sem_a)
    mk_wr(NCHUNK - 3, sc_a, semw_a).wait()
    compute(NCHUNK - 1, hr_a, tr_a, sw_a, sc_a)
    mk_wr(NCHUNK - 1, sc_a, semw_a).start()
    mk_wr(NCHUNK - 2, sc_b, semw_b).wait()
    mk_wr(NCHUNK - 1, sc_a, semw_a).wait()


def kernel(tuples, x, sw):
    tt = tuples.T
    u = jax.lax.bitcast_convert_type(x.astype(jnp.bfloat16), jnp.uint16)
    w = u.astype(jnp.uint32)
    xw = jax.lax.bitcast_convert_type(w[:, :DW] | (w[:, DW:] << 16),
                                      jnp.int32)
    return _score_kernel(tt, xw, sw)
